# baseline (speedup 1.0000x reference)
import functools

import jax
import jax.numpy as jnp
from jax import lax
from jax.experimental import pallas as pl
from jax.experimental.pallas import tpu as pltpu
from jax.experimental.pallas import tpu_sc as plsc
from jax.experimental import layout as jex_layout

_B = 16384
_NNUM = 13
_NCAT = 26
_V = 100000
_D = 64
_T = _NNUM + _NCAT

_NC = 2
_NS = 16
_NW = _NC * _NS
_RPW = _B // _NW
_NB = 16
_NCHUNK = _RPW // _NB


def _tokenizer_kernel(xnum_hbm, idx_hbm, hb_hbm, w_hbm, e_hbm, tables_hbm,
                      out_hbm, slab_v, idx_v, hb_v, xnum_v, w_v, e_v, stage_v,
                      sem):
    wid = lax.axis_index("s") * _NC + lax.axis_index("c")

    pltpu.sync_copy(w_hbm, w_v)
    pltpu.sync_copy(e_hbm, e_v)

    def chunk_body(c, carry):
        base = wid * _RPW + c * _NB

        pltpu.sync_copy(idx_hbm.at[pl.ds(base, _NB)], idx_v)
        pltpu.sync_copy(hb_hbm.at[pl.ds(base, _NB)], hb_v)
        pltpu.sync_copy(xnum_hbm.at[pl.ds(base * _NNUM, _NB * _NNUM)],
                        xnum_v.at[pl.ds(0, _NB * _NNUM)])

        copies = []
        for b in range(_NB):
            cp = pltpu.async_copy(
                tables_hbm.at[idx_v.at[b]],
                stage_v.at[pl.ds(b * _NCAT, _NCAT)],
                sem)
            copies.append(cp)

        for b in range(_NB):
            vrow = xnum_v[pl.ds(b * _NNUM, 16)]
            for j in range(_NNUM):
                sp = vrow[j]
                for q in range(_D // 16):
                    val = (sp * w_v[pl.ds(q * 16, 16)]
                           + e_v[pl.ds(j * _D + q * 16, 16)])
                    slab_v[b * _T + j, pl.ds(q * 16, 16)] = val

        for cp in copies:
            cp.wait()

        for b in range(_NB):
            ha = hb_v[b, pl.ds(0, 16)]
            hb2 = hb_v[b, pl.ds(10, 16)]
            for t in range(_NCAT):
                h = ha[t] if t < 16 else hb2[t - 10]
                off = h * _D
                row = b * _NCAT + t
                dst = b * _T + _NNUM + t
                for q in range(_D // 16):
                    slab_v[dst, pl.ds(q * 16, 16)] = (
                        stage_v[row, pl.ds(off + q * 16, 16)])

        pltpu.sync_copy(slab_v, out_hbm.at[pl.ds(base * _T, _NB * _T)])
        return carry

    lax.fori_loop(0, _NCHUNK, chunk_body, 0)


def kernel(x_num, x_cat, w_num, b_num, num_bias, tables):
    mesh = plsc.VectorSubcoreMesh(core_axis_name="c", subcore_axis_name="s")

    tp = tables.reshape(_NCAT * _V // 2, 2 * _D)
    tp = jex_layout.with_layout_constraint(
        tp, jex_layout.Layout(major_to_minor=(0, 1)))

    f_off = jnp.arange(_NCAT, dtype=jnp.int32)[None, :]
    flat = f_off * _V + x_cat
    pair = flat >> 1
    half = flat & 1
    e = (b_num[None, :] + num_bias).reshape(-1)

    k2 = pl.kernel(
        _tokenizer_kernel,
        out_type=jax.ShapeDtypeStruct((_B * _T, _D), jnp.float32),
        mesh=mesh,
        compiler_params=pltpu.CompilerParams(
            use_tc_tiling_on_sc=False, needs_layout_passes=False),
        scratch_types=[
            pltpu.VMEM((_NB * _T, _D), jnp.float32),
            pltpu.VMEM((_NB, _NCAT), jnp.int32),
            pltpu.VMEM((_NB, _NCAT), jnp.int32),
            pltpu.VMEM((_NB * _NNUM + 16,), jnp.float32),
            pltpu.VMEM((_D,), jnp.float32),
            pltpu.VMEM((_NNUM * _D,), jnp.float32),
            pltpu.VMEM((_NB * _NCAT, 2 * _D), jnp.float32),
            pltpu.SemaphoreType.DMA,
        ],
    )
    out = k2(x_num.reshape(-1), pair, half, w_num, e, tp)
    return out.reshape(_B, _T, _D)


# speedup vs baseline: 1.1570x; 1.1570x over previous
import functools

import jax
import jax.numpy as jnp
from jax import lax
from jax.experimental import pallas as pl
from jax.experimental.pallas import tpu as pltpu
from jax.experimental.pallas import tpu_sc as plsc

_B = 16384
_NNUM = 13
_NCAT = 26
_V = 100000
_D = 64
_T = _NNUM + _NCAT

_NC = 2
_NS = 16
_NW = _NC * _NS
_RPW = _B // _NW
_NB = 16
_NCHUNK = _RPW // _NB


def _tokenizer_kernel(xnum_hbm, idx_hbm, w_hbm, e_hbm, tables_hbm, out_hbm,
                      slab_v, idx_v, xnum_v, w_v, e_v, sem):
    wid = lax.axis_index("s") * _NC + lax.axis_index("c")

    pltpu.sync_copy(w_hbm, w_v)
    pltpu.sync_copy(e_hbm, e_v)

    def chunk_body(c, carry):
        base = wid * _RPW + c * _NB

        pltpu.sync_copy(idx_hbm.at[pl.ds(base, _NB)], idx_v)
        pltpu.sync_copy(xnum_hbm.at[pl.ds(base * _NNUM, _NB * _NNUM)],
                        xnum_v.at[pl.ds(0, _NB * _NNUM)])

        copies = []
        for b in range(_NB):
            cp = pltpu.async_copy(
                tables_hbm.at[idx_v.at[b]],
                slab_v.at[pl.ds(b * _T + _NNUM, _NCAT)],
                sem)
            copies.append(cp)

        for b in range(_NB):
            vrow = xnum_v[pl.ds(b * _NNUM, 16)]
            for j in range(_NNUM):
                sp = vrow[j]
                for q in range(_D // 16):
                    val = (sp * w_v[pl.ds(q * 16, 16)]
                           + e_v[pl.ds(j * _D + q * 16, 16)])
                    slab_v[b * _T + j, pl.ds(q * 16, 16)] = val

        for cp in copies:
            cp.wait()

        pltpu.sync_copy(slab_v, out_hbm.at[pl.ds(base * _T, _NB * _T)])
        return carry

    lax.fori_loop(0, _NCHUNK, chunk_body, 0)


def kernel(x_num, x_cat, w_num, b_num, num_bias, tables):
    flat_idx = x_cat + (jnp.arange(_NCAT, dtype=jnp.int32) * _V)[None, :]
    e = (b_num[None, :] + num_bias).reshape(-1)

    sc_call = pl.kernel(
        _tokenizer_kernel,
        out_type=jax.ShapeDtypeStruct((_B * _T, _D), jnp.float32),
        mesh=plsc.VectorSubcoreMesh(core_axis_name="c", subcore_axis_name="s"),
        compiler_params=pltpu.CompilerParams(use_tc_tiling_on_sc=False),
        scratch_types=[
            pltpu.VMEM((_NB * _T, _D), jnp.float32),
            pltpu.VMEM((_NB, _NCAT), jnp.int32),
            pltpu.VMEM((_NB * _NNUM + 16,), jnp.float32),
            pltpu.VMEM((_D,), jnp.float32),
            pltpu.VMEM((_NNUM * _D,), jnp.float32),
            pltpu.SemaphoreType.DMA,
        ],
    )
    out = sc_call(x_num.reshape(-1), flat_idx, w_num, e,
                  tables.reshape(_NCAT * _V, _D))
    return out.reshape(_B, _T, _D)
